# baseline (speedup 1.0000x reference)
import functools

import jax
import jax.numpy as jnp
from jax.experimental import pallas as pl
from jax.experimental.pallas import tpu as pltpu

B, C, S, N, D = 8, 16, 4, 256, 16
DLP = 16
DM = 16
NCLS = 2
K = S * N * DLP


def _gcn_block(adj_ref, feat_ref, w_ref, out_ref):
    adj = adj_ref[0, 0, 0]
    feat = feat_ref[0, 0, 0]
    w = w_ref[0, 0]
    fw = jnp.dot(feat, w, preferred_element_type=jnp.float32)
    h = jnp.dot(adj, fw, preferred_element_type=jnp.float32)
    out_ref[0, 0, 0] = jnp.maximum(h, 0.0)


def _head_block(z_ref, wm1_ref, wm2_ref, wdec_ref, bdec_ref, out_ref):
    rows = []
    for b in range(B):
        zb = z_ref[b]
        g = jax.lax.dot_general(zb, zb, (((1,), (1,)), ((), ())),
                                preferred_element_type=jnp.float32)
        row_i = jax.lax.broadcasted_iota(jnp.int32, (C, C), 0)
        col_i = jax.lax.broadcasted_iota(jnp.int32, (C, C), 1)
        eye = row_i == col_i
        diag = jnp.where(eye, g, 0.0)
        d_col = jnp.sqrt(jnp.sum(diag, axis=1, keepdims=True)) + 1e-8
        d_row = jnp.sqrt(jnp.sum(diag, axis=0, keepdims=True)) + 1e-8
        bg = g / (d_col * d_row)
        h1 = jnp.maximum(jnp.dot(bg, wm1_ref[...],
                                 preferred_element_type=jnp.float32), 0.0)
        t = jnp.dot(h1, wm2_ref[...], preferred_element_type=jnp.float32)
        h2 = jnp.maximum(jnp.dot(bg, t, preferred_element_type=jnp.float32), 0.0)
        dec = jnp.dot(h2, wdec_ref[...],
                      preferred_element_type=jnp.float32) + bdec_ref[...]
        pooled = jnp.sum(dec, axis=0, keepdims=True) * (1.0 / C)
        rows.append(pooled)
    out_ref[...] = jax.nn.sigmoid(jnp.concatenate(rows, axis=0))


@jax.jit
def kernel(feats, adjs, W_lp, W_m1, W_m2, W_dec, b_dec):
    emb = pl.pallas_call(
        _gcn_block,
        grid=(B, C, S),
        in_specs=[
            pl.BlockSpec((1, 1, 1, N, N), lambda b, c, s: (b, c, s, 0, 0)),
            pl.BlockSpec((1, 1, 1, N, D), lambda b, c, s: (b, c, s, 0, 0)),
            pl.BlockSpec((1, 1, D, DLP), lambda b, c, s: (c, s, 0, 0)),
        ],
        out_specs=pl.BlockSpec((1, 1, 1, N, DLP), lambda b, c, s: (b, c, s, 0, 0)),
        out_shape=jax.ShapeDtypeStruct((B, C, S, N, DLP), jnp.float32),
    )(adjs, feats, W_lp)

    z = emb.reshape(B, C, K)
    out = pl.pallas_call(
        _head_block,
        in_specs=[
            pl.BlockSpec((B, C, K), lambda: (0, 0, 0)),
            pl.BlockSpec((C, DM), lambda: (0, 0)),
            pl.BlockSpec((DM, DM), lambda: (0, 0)),
            pl.BlockSpec((DM, NCLS), lambda: (0, 0)),
            pl.BlockSpec((1, NCLS), lambda: (0, 0)),
        ],
        out_specs=pl.BlockSpec((B, NCLS), lambda: (0, 0)),
        out_shape=jax.ShapeDtypeStruct((B, NCLS), jnp.float32),
    )(z, W_m1, W_m2, W_dec, b_dec.reshape(1, NCLS))
    return out


# speedup vs baseline: 1.8042x; 1.8042x over previous
import functools

import jax
import jax.numpy as jnp
from jax.experimental import pallas as pl
from jax.experimental.pallas import tpu as pltpu

B, C, S, N, D = 8, 16, 4, 256, 16
DLP = 16
DM = 16
NCLS = 2
K = S * N * DLP


def _gcn_block(adj_ref, feat_ref, w_ref, out_ref):
    for s in range(S):
        adj = adj_ref[0, 0, s]
        feat = feat_ref[0, 0, s]
        w = w_ref[0, s]
        fw = jnp.dot(feat, w, preferred_element_type=jnp.float32)
        h = jnp.dot(adj, fw, preferred_element_type=jnp.float32)
        out_ref[0, 0, s] = jnp.maximum(h, 0.0)


def _head_block(z_ref, wm1_ref, wm2_ref, wdec_ref, bdec_ref, out_ref):
    rows = []
    for b in range(B):
        zb = z_ref[b]
        g = jax.lax.dot_general(zb, zb, (((1,), (1,)), ((), ())),
                                preferred_element_type=jnp.float32)
        row_i = jax.lax.broadcasted_iota(jnp.int32, (C, C), 0)
        col_i = jax.lax.broadcasted_iota(jnp.int32, (C, C), 1)
        eye = row_i == col_i
        diag = jnp.where(eye, g, 0.0)
        d_col = jnp.sqrt(jnp.sum(diag, axis=1, keepdims=True)) + 1e-8
        d_row = jnp.sqrt(jnp.sum(diag, axis=0, keepdims=True)) + 1e-8
        bg = g / (d_col * d_row)
        h1 = jnp.maximum(jnp.dot(bg, wm1_ref[...],
                                 preferred_element_type=jnp.float32), 0.0)
        t = jnp.dot(h1, wm2_ref[...], preferred_element_type=jnp.float32)
        h2 = jnp.maximum(jnp.dot(bg, t, preferred_element_type=jnp.float32), 0.0)
        dec = jnp.dot(h2, wdec_ref[...],
                      preferred_element_type=jnp.float32) + bdec_ref[...]
        pooled = jnp.sum(dec, axis=0, keepdims=True) * (1.0 / C)
        rows.append(pooled)
    out_ref[...] = jax.nn.sigmoid(jnp.concatenate(rows, axis=0))


@jax.jit
def kernel(feats, adjs, W_lp, W_m1, W_m2, W_dec, b_dec):
    emb = pl.pallas_call(
        _gcn_block,
        grid=(B, C),
        in_specs=[
            pl.BlockSpec((1, 1, S, N, N), lambda b, c: (b, c, 0, 0, 0)),
            pl.BlockSpec((1, 1, S, N, D), lambda b, c: (b, c, 0, 0, 0)),
            pl.BlockSpec((1, S, D, DLP), lambda b, c: (c, 0, 0, 0)),
        ],
        out_specs=pl.BlockSpec((1, 1, S, N, DLP), lambda b, c: (b, c, 0, 0, 0)),
        out_shape=jax.ShapeDtypeStruct((B, C, S, N, DLP), jnp.float32),
    )(adjs, feats, W_lp)

    z = emb.reshape(B, C, K)
    out = pl.pallas_call(
        _head_block,
        in_specs=[
            pl.BlockSpec((B, C, K), lambda: (0, 0, 0)),
            pl.BlockSpec((C, DM), lambda: (0, 0)),
            pl.BlockSpec((DM, DM), lambda: (0, 0)),
            pl.BlockSpec((DM, NCLS), lambda: (0, 0)),
            pl.BlockSpec((1, NCLS), lambda: (0, 0)),
        ],
        out_specs=pl.BlockSpec((B, NCLS), lambda: (0, 0)),
        out_shape=jax.ShapeDtypeStruct((B, NCLS), jnp.float32),
    )(z, W_m1, W_m2, W_dec, b_dec.reshape(1, NCLS))
    return out


# speedup vs baseline: 1.8067x; 1.0014x over previous
import functools

import jax
import jax.numpy as jnp
from jax.experimental import pallas as pl
from jax.experimental.pallas import tpu as pltpu

B, C, S, N, D = 8, 16, 4, 256, 16
DLP = 16
DM = 16
NCLS = 2
K = S * N * DLP


def _gcn_block(adj_ref, feat_ref, w_ref, out_ref):
    for s in range(S):
        adj = adj_ref[0, 0, s].astype(jnp.bfloat16)
        feat = feat_ref[0, 0, s]
        w = w_ref[0, s]
        fw = jnp.dot(feat, w, preferred_element_type=jnp.float32)
        h = jnp.dot(adj, fw.astype(jnp.bfloat16),
                    preferred_element_type=jnp.float32)
        out_ref[0, 0, s] = jnp.maximum(h, 0.0)


def _head_block(z_ref, wm1_ref, wm2_ref, wdec_ref, bdec_ref, out_ref):
    rows = []
    for b in range(B):
        zb = z_ref[b].astype(jnp.bfloat16)
        g = jax.lax.dot_general(zb, zb, (((1,), (1,)), ((), ())),
                                preferred_element_type=jnp.float32)
        row_i = jax.lax.broadcasted_iota(jnp.int32, (C, C), 0)
        col_i = jax.lax.broadcasted_iota(jnp.int32, (C, C), 1)
        eye = row_i == col_i
        diag = jnp.where(eye, g, 0.0)
        d_col = jnp.sqrt(jnp.sum(diag, axis=1, keepdims=True)) + 1e-8
        d_row = jnp.sqrt(jnp.sum(diag, axis=0, keepdims=True)) + 1e-8
        bg = g / (d_col * d_row)
        h1 = jnp.maximum(jnp.dot(bg, wm1_ref[...],
                                 preferred_element_type=jnp.float32), 0.0)
        t = jnp.dot(h1, wm2_ref[...], preferred_element_type=jnp.float32)
        h2 = jnp.maximum(jnp.dot(bg, t, preferred_element_type=jnp.float32), 0.0)
        dec = jnp.dot(h2, wdec_ref[...],
                      preferred_element_type=jnp.float32) + bdec_ref[...]
        pooled = jnp.sum(dec, axis=0, keepdims=True) * (1.0 / C)
        rows.append(pooled)
    out_ref[...] = jax.nn.sigmoid(jnp.concatenate(rows, axis=0))


@jax.jit
def kernel(feats, adjs, W_lp, W_m1, W_m2, W_dec, b_dec):
    emb = pl.pallas_call(
        _gcn_block,
        grid=(B, C),
        in_specs=[
            pl.BlockSpec((1, 1, S, N, N), lambda b, c: (b, c, 0, 0, 0)),
            pl.BlockSpec((1, 1, S, N, D), lambda b, c: (b, c, 0, 0, 0)),
            pl.BlockSpec((1, S, D, DLP), lambda b, c: (c, 0, 0, 0)),
        ],
        out_specs=pl.BlockSpec((1, 1, S, N, DLP), lambda b, c: (b, c, 0, 0, 0)),
        out_shape=jax.ShapeDtypeStruct((B, C, S, N, DLP), jnp.float32),
    )(adjs, feats, W_lp)

    z = emb.reshape(B, C, K)
    out = pl.pallas_call(
        _head_block,
        in_specs=[
            pl.BlockSpec((B, C, K), lambda: (0, 0, 0)),
            pl.BlockSpec((C, DM), lambda: (0, 0)),
            pl.BlockSpec((DM, DM), lambda: (0, 0)),
            pl.BlockSpec((DM, NCLS), lambda: (0, 0)),
            pl.BlockSpec((1, NCLS), lambda: (0, 0)),
        ],
        out_specs=pl.BlockSpec((B, NCLS), lambda: (0, 0)),
        out_shape=jax.ShapeDtypeStruct((B, NCLS), jnp.float32),
    )(z, W_m1, W_m2, W_dec, b_dec.reshape(1, NCLS))
    return out


# speedup vs baseline: 1.8093x; 1.0014x over previous
import jax
import jax.numpy as jnp
from jax.experimental import pallas as pl
from jax.experimental.pallas import tpu as pltpu

B, C, S, N, D = 8, 16, 4, 256, 16
DLP = 16
DM = 16
NCLS = 2
K = S * N * DLP


def _gcn_block(a0_ref, a1_ref, a2_ref, a3_ref, feat_ref, w_ref, out_ref):
    adj_refs = (a0_ref, a1_ref, a2_ref, a3_ref)
    for s in range(S):
        adj = adj_refs[s][0, 0, 0].astype(jnp.bfloat16)
        feat = feat_ref[0, 0, s]
        w = w_ref[0, s]
        fw = jnp.dot(feat, w, preferred_element_type=jnp.float32)
        h = jnp.dot(adj, fw.astype(jnp.bfloat16),
                    preferred_element_type=jnp.float32)
        out_ref[0, 0, s] = jnp.maximum(h, 0.0)


def _head_block(z_ref, wm1_ref, wm2_ref, wdec_ref, bdec_ref, out_ref):
    rows = []
    for b in range(B):
        zb = z_ref[b].astype(jnp.bfloat16)
        g = jax.lax.dot_general(zb, zb, (((1,), (1,)), ((), ())),
                                preferred_element_type=jnp.float32)
        row_i = jax.lax.broadcasted_iota(jnp.int32, (C, C), 0)
        col_i = jax.lax.broadcasted_iota(jnp.int32, (C, C), 1)
        eye = row_i == col_i
        diag = jnp.where(eye, g, 0.0)
        d_col = jnp.sqrt(jnp.sum(diag, axis=1, keepdims=True)) + 1e-8
        d_row = jnp.sqrt(jnp.sum(diag, axis=0, keepdims=True)) + 1e-8
        bg = g / (d_col * d_row)
        h1 = jnp.maximum(jnp.dot(bg, wm1_ref[...],
                                 preferred_element_type=jnp.float32), 0.0)
        t = jnp.dot(h1, wm2_ref[...], preferred_element_type=jnp.float32)
        h2 = jnp.maximum(jnp.dot(bg, t, preferred_element_type=jnp.float32), 0.0)
        dec = jnp.dot(h2, wdec_ref[...],
                      preferred_element_type=jnp.float32) + bdec_ref[...]
        pooled = jnp.sum(dec, axis=0, keepdims=True) * (1.0 / C)
        rows.append(pooled)
    out_ref[...] = jax.nn.sigmoid(jnp.concatenate(rows, axis=0))


@jax.jit
def kernel(feats, adjs, W_lp, W_m1, W_m2, W_dec, b_dec):
    adj_spec = lambda s: pl.BlockSpec((1, 1, 1, N, N),
                                      lambda b, c, s=s: (b, c, s, 0, 0))
    emb = pl.pallas_call(
        _gcn_block,
        grid=(B, C),
        in_specs=[adj_spec(0), adj_spec(1), adj_spec(2), adj_spec(3),
                  pl.BlockSpec((1, 1, S, N, D), lambda b, c: (b, c, 0, 0, 0)),
                  pl.BlockSpec((1, S, D, DLP), lambda b, c: (c, 0, 0, 0))],
        out_specs=pl.BlockSpec((1, 1, S, N, DLP), lambda b, c: (b, c, 0, 0, 0)),
        out_shape=jax.ShapeDtypeStruct((B, C, S, N, DLP), jnp.float32),
    )(adjs, adjs, adjs, adjs, feats, W_lp)

    z = emb.reshape(B, C, K)
    out = pl.pallas_call(
        _head_block,
        in_specs=[
            pl.BlockSpec((B, C, K), lambda: (0, 0, 0)),
            pl.BlockSpec((C, DM), lambda: (0, 0)),
            pl.BlockSpec((DM, DM), lambda: (0, 0)),
            pl.BlockSpec((DM, NCLS), lambda: (0, 0)),
            pl.BlockSpec((1, NCLS), lambda: (0, 0)),
        ],
        out_specs=pl.BlockSpec((B, NCLS), lambda: (0, 0)),
        out_shape=jax.ShapeDtypeStruct((B, NCLS), jnp.float32),
    )(z, W_m1, W_m2, W_dec, b_dec.reshape(1, NCLS))
    return out


# speedup vs baseline: 2.5397x; 1.4037x over previous
import jax
import jax.numpy as jnp
from jax.experimental import pallas as pl
from jax.experimental.pallas import tpu as pltpu

B, C, S, N, D = 8, 16, 4, 256, 16
DLP = 16
DM = 16
NCLS = 2
CD = C * DLP
SN = S * N


def _fused_block(a0_ref, a1_ref, a2_ref, a3_ref, feat_ref, w_ref,
                 wm1_ref, wm2_ref, wdec_ref, bdec_ref, out_ref, zm_ref):
    b = pl.program_id(0)
    c = pl.program_id(1)
    adj_refs = (a0_ref, a1_ref, a2_ref, a3_ref)
    for s in range(S):
        adj = adj_refs[s][0, 0, 0].astype(jnp.bfloat16)
        feat = feat_ref[0, 0, s]
        w = w_ref[0, s]
        fw = jnp.dot(feat, w, preferred_element_type=jnp.float32)
        h = jnp.dot(adj, fw.astype(jnp.bfloat16),
                    preferred_element_type=jnp.float32)
        ht = jnp.maximum(h, 0.0).astype(jnp.bfloat16).T
        zm_ref[pl.ds(c * DLP, DLP), pl.ds(s * N, N)] = ht

    @pl.when(c == C - 1)
    def _head():
        zm = zm_ref[...]
        q = jax.lax.dot_general(zm, zm, (((1,), (1,)), ((), ())),
                                preferred_element_type=jnp.float32)
        row_i = jax.lax.broadcasted_iota(jnp.int32, (CD, CD), 0)
        col_i = jax.lax.broadcasted_iota(jnp.int32, (CD, CD), 1)
        qm = jnp.where((row_i & (DLP - 1)) == (col_i & (DLP - 1)), q, 0.0)
        tt = (jax.lax.broadcasted_iota(jnp.int32, (C, CD), 1) // DLP
              == jax.lax.broadcasted_iota(jnp.int32, (C, CD), 0)
              ).astype(jnp.float32)
        a = jnp.dot(tt, qm, preferred_element_type=jnp.float32)
        g = jax.lax.dot_general(a, tt, (((1,), (1,)), ((), ())),
                                preferred_element_type=jnp.float32)
        row_c = jax.lax.broadcasted_iota(jnp.int32, (C, C), 0)
        col_c = jax.lax.broadcasted_iota(jnp.int32, (C, C), 1)
        diag = jnp.where(row_c == col_c, g, 0.0)
        d_col = jnp.sqrt(jnp.sum(diag, axis=1, keepdims=True)) + 1e-8
        d_row = jnp.sqrt(jnp.sum(diag, axis=0, keepdims=True)) + 1e-8
        bg = g / (d_col * d_row)
        h1 = jnp.maximum(jnp.dot(bg, wm1_ref[...],
                                 preferred_element_type=jnp.float32), 0.0)
        t = jnp.dot(h1, wm2_ref[...], preferred_element_type=jnp.float32)
        h2 = jnp.maximum(jnp.dot(bg, t, preferred_element_type=jnp.float32),
                         0.0)
        dec = jnp.dot(h2, wdec_ref[...],
                      preferred_element_type=jnp.float32) + bdec_ref[...]
        pooled = jnp.sum(dec, axis=0, keepdims=True) * (1.0 / C)
        out_ref[pl.ds(b, 1), :] = jax.nn.sigmoid(pooled)


@jax.jit
def kernel(feats, adjs, W_lp, W_m1, W_m2, W_dec, b_dec):
    adj_spec = lambda s: pl.BlockSpec((1, 1, 1, N, N),
                                      lambda b, c, s=s: (b, c, s, 0, 0))
    out = pl.pallas_call(
        _fused_block,
        grid=(B, C),
        in_specs=[adj_spec(0), adj_spec(1), adj_spec(2), adj_spec(3),
                  pl.BlockSpec((1, 1, S, N, D), lambda b, c: (b, c, 0, 0, 0)),
                  pl.BlockSpec((1, S, D, DLP), lambda b, c: (c, 0, 0, 0)),
                  pl.BlockSpec((C, DM), lambda b, c: (0, 0)),
                  pl.BlockSpec((DM, DM), lambda b, c: (0, 0)),
                  pl.BlockSpec((DM, NCLS), lambda b, c: (0, 0)),
                  pl.BlockSpec((1, NCLS), lambda b, c: (0, 0))],
        out_specs=pl.BlockSpec((B, NCLS), lambda b, c: (0, 0)),
        out_shape=jax.ShapeDtypeStruct((B, NCLS), jnp.float32),
        scratch_shapes=[pltpu.VMEM((CD, SN), jnp.bfloat16)],
    )(adjs, adjs, adjs, adjs, feats, W_lp, W_m1, W_m2, W_dec,
      b_dec.reshape(1, NCLS))
    return out


# speedup vs baseline: 3.1345x; 1.2342x over previous
import jax
import jax.numpy as jnp
from jax.experimental import pallas as pl
from jax.experimental.pallas import tpu as pltpu

B, C, S, N, D = 8, 16, 4, 256, 16
DLP = 16
DM = 16
NCLS = 2
CD = C * DLP
SN = S * N


CB = 2


def _fused_block(a0_ref, a1_ref, a2_ref, a3_ref, feat_ref, w_ref,
                 wm1_ref, wm2_ref, wdec_ref, bdec_ref, out_ref, zm_ref):
    b = pl.program_id(0)
    cg = pl.program_id(1)
    adj_refs = (a0_ref, a1_ref, a2_ref, a3_ref)
    for cb in range(CB):
        c = cg * CB + cb
        for s in range(S):
            adj = adj_refs[s][0, cb, 0].astype(jnp.bfloat16)
            feat = feat_ref[0, cb, s]
            w = w_ref[cb, s]
            fw = jnp.dot(feat, w, preferred_element_type=jnp.float32)
            h = jnp.dot(adj, fw.astype(jnp.bfloat16),
                        preferred_element_type=jnp.float32)
            ht = jnp.maximum(h, 0.0).astype(jnp.bfloat16).T
            zm_ref[pl.ds(c * DLP, DLP), pl.ds(s * N, N)] = ht

    @pl.when(cg == C // CB - 1)
    def _head():
        zm = zm_ref[...]
        q = jax.lax.dot_general(zm, zm, (((1,), (1,)), ((), ())),
                                preferred_element_type=jnp.float32)
        row_i = jax.lax.broadcasted_iota(jnp.int32, (CD, CD), 0)
        col_i = jax.lax.broadcasted_iota(jnp.int32, (CD, CD), 1)
        qm = jnp.where((row_i & (DLP - 1)) == (col_i & (DLP - 1)), q, 0.0)
        tt = (jax.lax.broadcasted_iota(jnp.int32, (C, CD), 1) // DLP
              == jax.lax.broadcasted_iota(jnp.int32, (C, CD), 0)
              ).astype(jnp.float32)
        a = jnp.dot(tt, qm, preferred_element_type=jnp.float32)
        g = jax.lax.dot_general(a, tt, (((1,), (1,)), ((), ())),
                                preferred_element_type=jnp.float32)
        row_c = jax.lax.broadcasted_iota(jnp.int32, (C, C), 0)
        col_c = jax.lax.broadcasted_iota(jnp.int32, (C, C), 1)
        diag = jnp.where(row_c == col_c, g, 0.0)
        d_col = jnp.sqrt(jnp.sum(diag, axis=1, keepdims=True)) + 1e-8
        d_row = jnp.sqrt(jnp.sum(diag, axis=0, keepdims=True)) + 1e-8
        bg = g / (d_col * d_row)
        h1 = jnp.maximum(jnp.dot(bg, wm1_ref[...],
                                 preferred_element_type=jnp.float32), 0.0)
        t = jnp.dot(h1, wm2_ref[...], preferred_element_type=jnp.float32)
        h2 = jnp.maximum(jnp.dot(bg, t, preferred_element_type=jnp.float32),
                         0.0)
        dec = jnp.dot(h2, wdec_ref[...],
                      preferred_element_type=jnp.float32) + bdec_ref[...]
        pooled = jnp.sum(dec, axis=0, keepdims=True) * (1.0 / C)
        out_ref[pl.ds(b, 1), :] = jax.nn.sigmoid(pooled)


@jax.jit
def kernel(feats, adjs, W_lp, W_m1, W_m2, W_dec, b_dec):
    adj_spec = lambda s: pl.BlockSpec((1, CB, 1, N, N),
                                      lambda b, c, s=s: (b, c, s, 0, 0))
    out = pl.pallas_call(
        _fused_block,
        grid=(B, C // CB),
        in_specs=[adj_spec(0), adj_spec(1), adj_spec(2), adj_spec(3),
                  pl.BlockSpec((1, CB, S, N, D), lambda b, c: (b, c, 0, 0, 0)),
                  pl.BlockSpec((CB, S, D, DLP), lambda b, c: (c, 0, 0, 0)),
                  pl.BlockSpec((C, DM), lambda b, c: (0, 0)),
                  pl.BlockSpec((DM, DM), lambda b, c: (0, 0)),
                  pl.BlockSpec((DM, NCLS), lambda b, c: (0, 0)),
                  pl.BlockSpec((1, NCLS), lambda b, c: (0, 0))],
        out_specs=pl.BlockSpec((B, NCLS), lambda b, c: (0, 0)),
        out_shape=jax.ShapeDtypeStruct((B, NCLS), jnp.float32),
        scratch_shapes=[pltpu.VMEM((CD, SN), jnp.bfloat16)],
    )(adjs, adjs, adjs, adjs, feats, W_lp, W_m1, W_m2, W_dec,
      b_dec.reshape(1, NCLS))
    return out


# speedup vs baseline: 3.5677x; 1.1382x over previous
import jax
import jax.numpy as jnp
from jax.experimental import pallas as pl
from jax.experimental.pallas import tpu as pltpu

B, C, S, N, D = 8, 16, 4, 256, 16
DLP = 16
DM = 16
NCLS = 2
CD = C * DLP
SN = S * N


CB = 4


def _fused_block(a0_ref, a1_ref, a2_ref, a3_ref, feat_ref, w_ref,
                 wm1_ref, wm2_ref, wdec_ref, bdec_ref, out_ref, zm_ref):
    b = pl.program_id(0)
    cg = pl.program_id(1)
    adj_refs = (a0_ref, a1_ref, a2_ref, a3_ref)
    for cb in range(CB):
        c = cg * CB + cb
        for s in range(S):
            adj = adj_refs[s][0, cb, 0].astype(jnp.bfloat16)
            feat = feat_ref[0, cb, s]
            w = w_ref[cb, s]
            fw = jnp.dot(feat, w, preferred_element_type=jnp.float32)
            h = jnp.dot(adj, fw.astype(jnp.bfloat16),
                        preferred_element_type=jnp.float32)
            ht = jnp.maximum(h, 0.0).astype(jnp.bfloat16).T
            zm_ref[pl.ds(c * DLP, DLP), pl.ds(s * N, N)] = ht

    @pl.when(cg == C // CB - 1)
    def _head():
        zm = zm_ref[...]
        q = jax.lax.dot_general(zm, zm, (((1,), (1,)), ((), ())),
                                preferred_element_type=jnp.float32)
        row_i = jax.lax.broadcasted_iota(jnp.int32, (CD, CD), 0)
        col_i = jax.lax.broadcasted_iota(jnp.int32, (CD, CD), 1)
        qm = jnp.where((row_i & (DLP - 1)) == (col_i & (DLP - 1)), q, 0.0)
        tt = (jax.lax.broadcasted_iota(jnp.int32, (C, CD), 1) // DLP
              == jax.lax.broadcasted_iota(jnp.int32, (C, CD), 0)
              ).astype(jnp.float32)
        a = jnp.dot(tt, qm, preferred_element_type=jnp.float32)
        g = jax.lax.dot_general(a, tt, (((1,), (1,)), ((), ())),
                                preferred_element_type=jnp.float32)
        row_c = jax.lax.broadcasted_iota(jnp.int32, (C, C), 0)
        col_c = jax.lax.broadcasted_iota(jnp.int32, (C, C), 1)
        diag = jnp.where(row_c == col_c, g, 0.0)
        d_col = jnp.sqrt(jnp.sum(diag, axis=1, keepdims=True)) + 1e-8
        d_row = jnp.sqrt(jnp.sum(diag, axis=0, keepdims=True)) + 1e-8
        bg = g / (d_col * d_row)
        h1 = jnp.maximum(jnp.dot(bg, wm1_ref[...],
                                 preferred_element_type=jnp.float32), 0.0)
        t = jnp.dot(h1, wm2_ref[...], preferred_element_type=jnp.float32)
        h2 = jnp.maximum(jnp.dot(bg, t, preferred_element_type=jnp.float32),
                         0.0)
        dec = jnp.dot(h2, wdec_ref[...],
                      preferred_element_type=jnp.float32) + bdec_ref[...]
        pooled = jnp.sum(dec, axis=0, keepdims=True) * (1.0 / C)
        out_ref[pl.ds(b, 1), :] = jax.nn.sigmoid(pooled)


@jax.jit
def kernel(feats, adjs, W_lp, W_m1, W_m2, W_dec, b_dec):
    adj_spec = lambda s: pl.BlockSpec((1, CB, 1, N, N),
                                      lambda b, c, s=s: (b, c, s, 0, 0))
    out = pl.pallas_call(
        _fused_block,
        grid=(B, C // CB),
        in_specs=[adj_spec(0), adj_spec(1), adj_spec(2), adj_spec(3),
                  pl.BlockSpec((1, CB, S, N, D), lambda b, c: (b, c, 0, 0, 0)),
                  pl.BlockSpec((CB, S, D, DLP), lambda b, c: (c, 0, 0, 0)),
                  pl.BlockSpec((C, DM), lambda b, c: (0, 0)),
                  pl.BlockSpec((DM, DM), lambda b, c: (0, 0)),
                  pl.BlockSpec((DM, NCLS), lambda b, c: (0, 0)),
                  pl.BlockSpec((1, NCLS), lambda b, c: (0, 0))],
        out_specs=pl.BlockSpec((B, NCLS), lambda b, c: (0, 0)),
        out_shape=jax.ShapeDtypeStruct((B, NCLS), jnp.float32),
        scratch_shapes=[pltpu.VMEM((CD, SN), jnp.bfloat16)],
    )(adjs, adjs, adjs, adjs, feats, W_lp, W_m1, W_m2, W_dec,
      b_dec.reshape(1, NCLS))
    return out


# speedup vs baseline: 3.7749x; 1.0581x over previous
import jax
import jax.numpy as jnp
from jax.experimental import pallas as pl
from jax.experimental.pallas import tpu as pltpu

B, C, S, N, D = 8, 16, 4, 256, 16
DLP = 16
DM = 16
NCLS = 2
CD = C * DLP
SN = S * N


CB = 8


def _fused_block(a0_ref, a1_ref, a2_ref, a3_ref, feat_ref, w_ref,
                 wm1_ref, wm2_ref, wdec_ref, bdec_ref, out_ref, zm_ref):
    b = pl.program_id(0)
    cg = pl.program_id(1)
    adj_refs = (a0_ref, a1_ref, a2_ref, a3_ref)
    for cb in range(CB):
        c = cg * CB + cb
        for s in range(S):
            adj = adj_refs[s][0, cb, 0].astype(jnp.bfloat16)
            feat = feat_ref[0, cb, s]
            w = w_ref[cb, s]
            fw = jnp.dot(feat, w, preferred_element_type=jnp.float32)
            h = jnp.dot(adj, fw.astype(jnp.bfloat16),
                        preferred_element_type=jnp.float32)
            ht = jnp.maximum(h, 0.0).astype(jnp.bfloat16).T
            zm_ref[pl.ds(c * DLP, DLP), pl.ds(s * N, N)] = ht

    @pl.when(cg == C // CB - 1)
    def _head():
        zm = zm_ref[...]
        q = jax.lax.dot_general(zm, zm, (((1,), (1,)), ((), ())),
                                preferred_element_type=jnp.float32)
        row_i = jax.lax.broadcasted_iota(jnp.int32, (CD, CD), 0)
        col_i = jax.lax.broadcasted_iota(jnp.int32, (CD, CD), 1)
        qm = jnp.where((row_i & (DLP - 1)) == (col_i & (DLP - 1)), q, 0.0)
        tt = (jax.lax.broadcasted_iota(jnp.int32, (C, CD), 1) // DLP
              == jax.lax.broadcasted_iota(jnp.int32, (C, CD), 0)
              ).astype(jnp.float32)
        a = jnp.dot(tt, qm, preferred_element_type=jnp.float32)
        g = jax.lax.dot_general(a, tt, (((1,), (1,)), ((), ())),
                                preferred_element_type=jnp.float32)
        row_c = jax.lax.broadcasted_iota(jnp.int32, (C, C), 0)
        col_c = jax.lax.broadcasted_iota(jnp.int32, (C, C), 1)
        diag = jnp.where(row_c == col_c, g, 0.0)
        d_col = jnp.sqrt(jnp.sum(diag, axis=1, keepdims=True)) + 1e-8
        d_row = jnp.sqrt(jnp.sum(diag, axis=0, keepdims=True)) + 1e-8
        bg = g / (d_col * d_row)
        h1 = jnp.maximum(jnp.dot(bg, wm1_ref[...],
                                 preferred_element_type=jnp.float32), 0.0)
        t = jnp.dot(h1, wm2_ref[...], preferred_element_type=jnp.float32)
        h2 = jnp.maximum(jnp.dot(bg, t, preferred_element_type=jnp.float32),
                         0.0)
        dec = jnp.dot(h2, wdec_ref[...],
                      preferred_element_type=jnp.float32) + bdec_ref[...]
        pooled = jnp.sum(dec, axis=0, keepdims=True) * (1.0 / C)
        out_ref[pl.ds(b, 1), :] = jax.nn.sigmoid(pooled)


@jax.jit
def kernel(feats, adjs, W_lp, W_m1, W_m2, W_dec, b_dec):
    adj_spec = lambda s: pl.BlockSpec((1, CB, 1, N, N),
                                      lambda b, c, s=s: (b, c, s, 0, 0))
    out = pl.pallas_call(
        _fused_block,
        grid=(B, C // CB),
        in_specs=[adj_spec(0), adj_spec(1), adj_spec(2), adj_spec(3),
                  pl.BlockSpec((1, CB, S, N, D), lambda b, c: (b, c, 0, 0, 0)),
                  pl.BlockSpec((CB, S, D, DLP), lambda b, c: (c, 0, 0, 0)),
                  pl.BlockSpec((C, DM), lambda b, c: (0, 0)),
                  pl.BlockSpec((DM, DM), lambda b, c: (0, 0)),
                  pl.BlockSpec((DM, NCLS), lambda b, c: (0, 0)),
                  pl.BlockSpec((1, NCLS), lambda b, c: (0, 0))],
        out_specs=pl.BlockSpec((B, NCLS), lambda b, c: (0, 0)),
        out_shape=jax.ShapeDtypeStruct((B, NCLS), jnp.float32),
        scratch_shapes=[pltpu.VMEM((CD, SN), jnp.bfloat16)],
        compiler_params=pltpu.CompilerParams(vmem_limit_bytes=120 * 1024 * 1024),
    )(adjs, adjs, adjs, adjs, feats, W_lp, W_m1, W_m2, W_dec,
      b_dec.reshape(1, NCLS))
    return out


# speedup vs baseline: 3.8809x; 1.0281x over previous
import jax
import jax.numpy as jnp
from jax.experimental import pallas as pl
from jax.experimental.pallas import tpu as pltpu

B, C, S, N, D = 8, 16, 4, 256, 16
DLP = 16
DM = 16
NCLS = 2
CD = C * DLP
SN = S * N


CB = 16


def _fused_block(a0_ref, a1_ref, a2_ref, a3_ref, feat_ref, w_ref,
                 wm1_ref, wm2_ref, wdec_ref, bdec_ref, out_ref, zm_ref):
    b = pl.program_id(0)
    cg = pl.program_id(1)
    adj_refs = (a0_ref, a1_ref, a2_ref, a3_ref)
    for cb in range(CB):
        c = cg * CB + cb
        for s in range(S):
            adj = adj_refs[s][0, cb, 0].astype(jnp.bfloat16)
            feat = feat_ref[0, cb, s]
            w = w_ref[cb, s]
            fw = jnp.dot(feat, w, preferred_element_type=jnp.float32)
            h = jnp.dot(adj, fw.astype(jnp.bfloat16),
                        preferred_element_type=jnp.float32)
            ht = jnp.maximum(h, 0.0).astype(jnp.bfloat16).T
            zm_ref[pl.ds(c * DLP, DLP), pl.ds(s * N, N)] = ht

    @pl.when(cg == C // CB - 1)
    def _head():
        zm = zm_ref[...]
        q = jax.lax.dot_general(zm, zm, (((1,), (1,)), ((), ())),
                                preferred_element_type=jnp.float32)
        row_i = jax.lax.broadcasted_iota(jnp.int32, (CD, CD), 0)
        col_i = jax.lax.broadcasted_iota(jnp.int32, (CD, CD), 1)
        qm = jnp.where((row_i & (DLP - 1)) == (col_i & (DLP - 1)), q, 0.0)
        tt = (jax.lax.broadcasted_iota(jnp.int32, (C, CD), 1) // DLP
              == jax.lax.broadcasted_iota(jnp.int32, (C, CD), 0)
              ).astype(jnp.float32)
        a = jnp.dot(tt, qm, preferred_element_type=jnp.float32)
        g = jax.lax.dot_general(a, tt, (((1,), (1,)), ((), ())),
                                preferred_element_type=jnp.float32)
        row_c = jax.lax.broadcasted_iota(jnp.int32, (C, C), 0)
        col_c = jax.lax.broadcasted_iota(jnp.int32, (C, C), 1)
        diag = jnp.where(row_c == col_c, g, 0.0)
        d_col = jnp.sqrt(jnp.sum(diag, axis=1, keepdims=True)) + 1e-8
        d_row = jnp.sqrt(jnp.sum(diag, axis=0, keepdims=True)) + 1e-8
        bg = g / (d_col * d_row)
        h1 = jnp.maximum(jnp.dot(bg, wm1_ref[...],
                                 preferred_element_type=jnp.float32), 0.0)
        t = jnp.dot(h1, wm2_ref[...], preferred_element_type=jnp.float32)
        h2 = jnp.maximum(jnp.dot(bg, t, preferred_element_type=jnp.float32),
                         0.0)
        dec = jnp.dot(h2, wdec_ref[...],
                      preferred_element_type=jnp.float32) + bdec_ref[...]
        pooled = jnp.sum(dec, axis=0, keepdims=True) * (1.0 / C)
        out_ref[pl.ds(b, 1), :] = jax.nn.sigmoid(pooled)


@jax.jit
def kernel(feats, adjs, W_lp, W_m1, W_m2, W_dec, b_dec):
    adj_spec = lambda s: pl.BlockSpec((1, CB, 1, N, N),
                                      lambda b, c, s=s: (b, c, s, 0, 0))
    out = pl.pallas_call(
        _fused_block,
        grid=(B, C // CB),
        in_specs=[adj_spec(0), adj_spec(1), adj_spec(2), adj_spec(3),
                  pl.BlockSpec((1, CB, S, N, D), lambda b, c: (b, c, 0, 0, 0)),
                  pl.BlockSpec((CB, S, D, DLP), lambda b, c: (c, 0, 0, 0)),
                  pl.BlockSpec((C, DM), lambda b, c: (0, 0)),
                  pl.BlockSpec((DM, DM), lambda b, c: (0, 0)),
                  pl.BlockSpec((DM, NCLS), lambda b, c: (0, 0)),
                  pl.BlockSpec((1, NCLS), lambda b, c: (0, 0))],
        out_specs=pl.BlockSpec((B, NCLS), lambda b, c: (0, 0)),
        out_shape=jax.ShapeDtypeStruct((B, NCLS), jnp.float32),
        scratch_shapes=[pltpu.VMEM((CD, SN), jnp.bfloat16)],
        compiler_params=pltpu.CompilerParams(vmem_limit_bytes=120 * 1024 * 1024),
    )(adjs, adjs, adjs, adjs, feats, W_lp, W_m1, W_m2, W_dec,
      b_dec.reshape(1, NCLS))
    return out
